# Initial kernel scaffold; baseline (speedup 1.0000x reference)
import jax
import jax.numpy as jnp
from jax.experimental import pallas as pl


def kernel(X, A, idx):
    raise NotImplementedError("write your pallas kernel here")


# speedup vs baseline: 1.1720x; 1.1720x over previous
import functools

import jax
import jax.numpy as jnp
from jax import lax
from jax.experimental import pallas as pl
from jax.experimental.pallas import tpu as pltpu
from jax.experimental.pallas import tpu_sc as plsc


def _build_scatter(B, N_old, F, N_new):
    info = plsc.get_sparse_core_info()
    NC, NS, L = info.num_cores, info.num_subcores, info.num_lanes
    assert B % NC == 0 and N_old % NS == 0 and N_new % NS == 0 and F % L == 0
    BPC = B // NC
    RPT = N_old // NS
    OPT = N_new // NS
    ZR = 64
    assert OPT % ZR == 0

    mesh = plsc.VectorSubcoreMesh(core_axis_name="c", subcore_axis_name="s")

    @functools.partial(
        pl.kernel,
        mesh=mesh,
        out_type=jax.ShapeDtypeStruct((B * N_new, F), jnp.float32),
        compiler_params=pltpu.CompilerParams(use_tc_tiling_on_sc=False),
        scratch_types=[
            pltpu.VMEM_SHARED((N_new, F), jnp.float32),
            pltpu.VMEM((ZR, F), jnp.float32),
            pltpu.VMEM((RPT,), jnp.int32),
            pltpu.VMEM((RPT, F), jnp.float32),
        ],
    )
    def scatter_kernel(x_hbm, idx_hbm, out_hbm, acc, zbuf, idxv, rowsv):
        c = lax.axis_index("c")
        s = lax.axis_index("s")
        zv = jnp.zeros((L,), jnp.float32)

        def zstore(k, _):
            r = k // (F // L)
            off = (k % (F // L)) * L
            zbuf[r, pl.ds(off, L)] = zv
            return 0
        lax.fori_loop(0, ZR * (F // L), zstore, 0)

        for p in range(BPC):
            b = c * BPC + p
            for j in range(OPT // ZR):
                pltpu.sync_copy(zbuf, acc.at[pl.ds(s * OPT + j * ZR, ZR)])
            plsc.subcore_barrier()
            pltpu.sync_copy(idx_hbm.at[b, pl.ds(s * RPT, RPT)], idxv)
            pltpu.sync_copy(x_hbm.at[pl.ds(b * N_old + s * RPT, RPT)], rowsv)
            pltpu.sync_copy(rowsv, acc.at[idxv.at[:]], add=True)
            plsc.subcore_barrier()
            pltpu.sync_copy(
                acc.at[pl.ds(s * OPT, OPT)],
                out_hbm.at[pl.ds(b * N_new + s * OPT, OPT)],
            )

    return scatter_kernel


def kernel(X, A, idx):
    B, N_old, F = X.shape
    N_new = A.shape[1]
    X_flat = X.reshape(B * N_old, F)
    idx2 = idx.reshape(B, N_old)
    new_X = _build_scatter(B, N_old, F, N_new)(X_flat, idx2)
    return (new_X.reshape(B, N_new, F), A)
